# baseline (speedup 1.0000x reference)
import jax
import jax.numpy as jnp
from jax import lax
from jax.experimental import pallas as pl
from jax.experimental.pallas import tpu as pltpu
from jax.experimental.pallas import tpu_sc as plsc

_L = 16


def _sc_body(I_hbm, d_hbm, zl_hbm, o1_hbm, o2_hbm,
             im_v, d_v, zl_v, o1_v, o2_v, sem):
    wid = lax.axis_index("s") * 2 + lax.axis_index("c")
    c1 = pltpu.async_copy(I_hbm.at[wid], im_v, sem)
    c2 = pltpu.async_copy(d_hbm.at[wid], d_v, sem)
    c3 = pltpu.async_copy(zl_hbm.at[wid], zl_v, sem)
    c1.wait()
    c2.wait()
    c3.wait()

    zero = jnp.zeros((_L,), jnp.float32)

    def group(g0, carry):
        g = g0 * _L
        m = im_v[pl.ds(g, _L)].astype(jnp.float32)
        zlI = zl_v[pl.ds(g, _L)] * m
        nd = -d_v[pl.ds(g, _L)]
        o1_v[pl.ds(g, _L)] = zlI * jnp.maximum(nd, zero)
        o2_v[pl.ds(g, _L)] = zlI * jnp.minimum(nd, zero)
        return carry

    lax.fori_loop(0, 128, group, 0)
    pltpu.sync_copy(o1_v, o1_hbm.at[wid])
    pltpu.sync_copy(o2_v, o2_hbm.at[wid])


def kernel(I, d, zl):
    B, n = d.shape
    out = jax.ShapeDtypeStruct((B, n), jnp.float32)
    mesh = plsc.VectorSubcoreMesh(
        core_axis_name="c", subcore_axis_name="s",
        num_cores=2, num_subcores=16,
    )
    f = pl.kernel(
        _sc_body,
        out_type=(out, out),
        mesh=mesh,
        scratch_types=[
            pltpu.VMEM((n,), jnp.int32),
            pltpu.VMEM((n,), jnp.float32),
            pltpu.VMEM((n,), jnp.float32),
            pltpu.VMEM((n,), jnp.float32),
            pltpu.VMEM((n,), jnp.float32),
            pltpu.SemaphoreType.DMA,
        ],
    )
    return f(I, d, zl)


# speedup vs baseline: 5.1450x; 5.1450x over previous
import jax
import jax.numpy as jnp
from jax.experimental import pallas as pl
from jax.experimental.pallas import tpu as pltpu


def _body(I_ref, d_ref, zl_ref, o_zl_ref, o_zu_ref):
    m = I_ref[...].astype(jnp.float32)
    dI = d_ref[...] * m
    zlI = zl_ref[...] * m
    o_zl_ref[...] = zlI * jnp.maximum(-dI, 0.0)
    o_zu_ref[...] = -(zlI * jnp.maximum(dI, 0.0))


def kernel(I, d, zl):
    B, n = d.shape
    out = jax.ShapeDtypeStruct((B, n), jnp.float32)
    spec = pl.BlockSpec(memory_space=pltpu.VMEM)
    return pl.pallas_call(
        _body,
        out_shape=(out, out),
        in_specs=[spec, spec, spec],
        out_specs=(spec, spec),
    )(I, d, zl)
